# baseline (speedup 1.0000x reference)
import functools

import jax
import jax.numpy as jnp
from jax.experimental import pallas as pl
from jax.experimental.pallas import tpu as pltpu

N = 4096
D = 128
TEMP = 0.1
K = max(1, int(0.5 * (N - 1)))
BLOCK = 1024
N_ITERS = 12
LO0 = -1.00005 / TEMP
HI0 = 1.00005 / TEMP


def _normalize_kernel(q_ref, p_ref, qn_ref, pn_ref):
    q = q_ref[...]
    p = p_ref[...]
    qs = (1.0 / TEMP) / jnp.maximum(
        jnp.sqrt(jnp.sum(q * q, axis=1, keepdims=True)), 1e-12)
    ps = 1.0 / jnp.maximum(
        jnp.sqrt(jnp.sum(p * p, axis=1, keepdims=True)), 1e-12)
    qn_ref[...] = q * qs
    pn_ref[...] = p * ps


def _loss_block_kernel(q_ref, p_ref, pblk_ref, out_ref):
    qb = q_ref[...]
    pf = p_ref[...]

    logits = jax.lax.dot_general(
        qb, pf, dimension_numbers=(((1,), (1,)), ((), ())),
        preferred_element_type=jnp.float32,
    )

    pos = jnp.sum(qb * pblk_ref[...], axis=1, keepdims=True)

    kf = jnp.float32(K)

    def bs_body(_, carry):
        lo, hi, clo, chi = carry
        mid = 0.5 * (lo + hi)
        cnt = jnp.sum((logits < mid).astype(jnp.float32), axis=1, keepdims=True)
        cnt = cnt - (pos < mid).astype(jnp.float32)
        ge = cnt >= kf
        return (jnp.where(ge, lo, mid), jnp.where(ge, mid, hi),
                jnp.where(ge, clo, cnt), jnp.where(ge, cnt, chi))

    lo = jnp.full((BLOCK, 1), LO0, jnp.float32)
    hi = jnp.full((BLOCK, 1), HI0, jnp.float32)
    clo = jnp.zeros((BLOCK, 1), jnp.float32)
    chi = jnp.full((BLOCK, 1), float(N - 1), jnp.float32)
    lo, hi, clo, chi = jax.lax.fori_loop(
        0, N_ITERS, bs_body, (lo, hi, clo, chi))
    t = lo + (kf - clo) / (chi - clo) * (hi - lo)

    ex = jnp.exp(jnp.minimum(logits, t))
    s = jnp.sum(ex, axis=1, keepdims=True)
    s = (s - jnp.exp(jnp.minimum(pos, t))
         - (N - 1 - K) * jnp.exp(t) + jnp.exp(pos))
    losses = -pos + jnp.log(s)

    out_ref[...] = jnp.sum(losses).reshape(1, 1, 1)


@jax.jit
def kernel(query, positive_key):
    qn, pn = pl.pallas_call(
        _normalize_kernel,
        in_specs=[
            pl.BlockSpec((N, D), lambda: (0, 0)),
            pl.BlockSpec((N, D), lambda: (0, 0)),
        ],
        out_specs=[
            pl.BlockSpec((N, D), lambda: (0, 0)),
            pl.BlockSpec((N, D), lambda: (0, 0)),
        ],
        out_shape=[
            jax.ShapeDtypeStruct((N, D), jnp.float32),
            jax.ShapeDtypeStruct((N, D), jnp.float32),
        ],
    )(query, positive_key)

    out = pl.pallas_call(
        _loss_block_kernel,
        grid=(N // BLOCK,),
        in_specs=[
            pl.BlockSpec((BLOCK, D), lambda i: (i, 0)),
            pl.BlockSpec((N, D), lambda i: (0, 0)),
            pl.BlockSpec((BLOCK, D), lambda i: (i, 0)),
        ],
        out_specs=pl.BlockSpec((1, 1, 1), lambda i: (i, 0, 0)),
        out_shape=jax.ShapeDtypeStruct((N // BLOCK, 1, 1), jnp.float32),
        compiler_params=pltpu.CompilerParams(
            dimension_semantics=("parallel",),
        ),
    )(qn, pn, pn)
    return jnp.sum(out) / N


# speedup vs baseline: 1.0649x; 1.0649x over previous
import functools

import jax
import jax.numpy as jnp
from jax.experimental import pallas as pl
from jax.experimental.pallas import tpu as pltpu

N = 4096
D = 128
TEMP = 0.1
K = max(1, int(0.5 * (N - 1)))
BLOCK = 1024
N_ITERS = 13
LO0 = -1.00005 / TEMP
HI0 = 1.00005 / TEMP


def _normalize_kernel(q_ref, p_ref, qn_ref, pn_ref):
    q = q_ref[...]
    p = p_ref[...]
    qs = (1.0 / TEMP) / jnp.maximum(
        jnp.sqrt(jnp.sum(q * q, axis=1, keepdims=True)), 1e-12)
    ps = 1.0 / jnp.maximum(
        jnp.sqrt(jnp.sum(p * p, axis=1, keepdims=True)), 1e-12)
    qn_ref[...] = q * qs
    pn_ref[...] = p * ps


def _loss_block_kernel(q_ref, p_ref, pblk_ref, out_ref):
    qb = q_ref[...]
    pf = p_ref[...]

    logits = jax.lax.dot_general(
        qb, pf, dimension_numbers=(((1,), (1,)), ((), ())),
        preferred_element_type=jnp.float32,
    )

    pos = jnp.sum(qb * pblk_ref[...], axis=1, keepdims=True)

    kf = jnp.float32(K)

    def bs_body(_, carry):
        lo, hi = carry
        mid = 0.5 * (lo + hi)
        cnt = jnp.sum((logits < mid).astype(jnp.float32), axis=1, keepdims=True)
        cnt = cnt - (pos < mid).astype(jnp.float32)
        ge = cnt >= kf
        return jnp.where(ge, lo, mid), jnp.where(ge, mid, hi)

    lo = jnp.full((BLOCK, 1), LO0, jnp.float32)
    hi = jnp.full((BLOCK, 1), HI0, jnp.float32)
    lo, hi = jax.lax.fori_loop(0, N_ITERS, bs_body, (lo, hi))
    t = 0.5 * (lo + hi)

    ex = jnp.exp(jnp.minimum(logits, t))
    s = jnp.sum(ex, axis=1, keepdims=True)
    s = (s - jnp.exp(jnp.minimum(pos, t))
         - (N - 1 - K) * jnp.exp(t) + jnp.exp(pos))
    losses = -pos + jnp.log(s)

    out_ref[...] = jnp.sum(losses).reshape(1, 1, 1)


@jax.jit
def kernel(query, positive_key):
    qn, pn = pl.pallas_call(
        _normalize_kernel,
        in_specs=[
            pl.BlockSpec((N, D), lambda: (0, 0)),
            pl.BlockSpec((N, D), lambda: (0, 0)),
        ],
        out_specs=[
            pl.BlockSpec((N, D), lambda: (0, 0)),
            pl.BlockSpec((N, D), lambda: (0, 0)),
        ],
        out_shape=[
            jax.ShapeDtypeStruct((N, D), jnp.float32),
            jax.ShapeDtypeStruct((N, D), jnp.float32),
        ],
    )(query, positive_key)

    out = pl.pallas_call(
        _loss_block_kernel,
        grid=(N // BLOCK,),
        in_specs=[
            pl.BlockSpec((BLOCK, D), lambda i: (i, 0)),
            pl.BlockSpec((N, D), lambda i: (0, 0)),
            pl.BlockSpec((BLOCK, D), lambda i: (i, 0)),
        ],
        out_specs=pl.BlockSpec((1, 1, 1), lambda i: (i, 0, 0)),
        out_shape=jax.ShapeDtypeStruct((N // BLOCK, 1, 1), jnp.float32),
        compiler_params=pltpu.CompilerParams(
            dimension_semantics=("parallel",),
        ),
    )(qn, pn, pn)
    return jnp.sum(out) / N


# speedup vs baseline: 1.0702x; 1.0050x over previous
import functools

import jax
import jax.numpy as jnp
from jax.experimental import pallas as pl
from jax.experimental.pallas import tpu as pltpu

N = 4096
D = 128
TEMP = 0.1
K = max(1, int(0.5 * (N - 1)))
BLOCK = 1024
N_ITERS = 13
LOG2E = 1.4426950408889634
LN2 = 0.6931471805599453
LO0 = -1.00005 * LOG2E / TEMP
HI0 = 1.00005 * LOG2E / TEMP


def _normalize_kernel(q_ref, p_ref, qn_ref, pn_ref):
    q = q_ref[...]
    p = p_ref[...]
    qs = (LOG2E / TEMP) / jnp.maximum(
        jnp.sqrt(jnp.sum(q * q, axis=1, keepdims=True)), 1e-12)
    ps = 1.0 / jnp.maximum(
        jnp.sqrt(jnp.sum(p * p, axis=1, keepdims=True)), 1e-12)
    qn_ref[...] = q * qs
    pn_ref[...] = p * ps


def _loss_block_kernel(q_ref, p_ref, pblk_ref, out_ref):
    qb = q_ref[...]
    pf = p_ref[...]

    logits = jax.lax.dot_general(
        qb, pf, dimension_numbers=(((1,), (1,)), ((), ())),
        preferred_element_type=jnp.float32,
    )

    pos = jnp.sum(qb * pblk_ref[...], axis=1, keepdims=True)

    kf = jnp.float32(K)

    def bs_body(_, carry):
        lo, hi = carry
        mid = 0.5 * (lo + hi)
        cnt = jnp.sum((logits < mid).astype(jnp.float32), axis=1, keepdims=True)
        cnt = cnt - (pos < mid).astype(jnp.float32)
        ge = cnt >= kf
        return jnp.where(ge, lo, mid), jnp.where(ge, mid, hi)

    lo = jnp.full((BLOCK, 1), LO0, jnp.float32)
    hi = jnp.full((BLOCK, 1), HI0, jnp.float32)
    lo, hi = jax.lax.fori_loop(0, N_ITERS, bs_body, (lo, hi))
    t = 0.5 * (lo + hi)

    ex = jnp.exp2(jnp.minimum(logits, t))
    s = jnp.sum(ex, axis=1, keepdims=True)
    s = (s - jnp.exp2(jnp.minimum(pos, t))
         - (N - 1 - K) * jnp.exp2(t) + jnp.exp2(pos))
    losses = -pos * LN2 + jnp.log(s)

    out_ref[...] = jnp.sum(losses).reshape(1, 1, 1)


@jax.jit
def kernel(query, positive_key):
    qn, pn = pl.pallas_call(
        _normalize_kernel,
        in_specs=[
            pl.BlockSpec((N, D), lambda: (0, 0)),
            pl.BlockSpec((N, D), lambda: (0, 0)),
        ],
        out_specs=[
            pl.BlockSpec((N, D), lambda: (0, 0)),
            pl.BlockSpec((N, D), lambda: (0, 0)),
        ],
        out_shape=[
            jax.ShapeDtypeStruct((N, D), jnp.float32),
            jax.ShapeDtypeStruct((N, D), jnp.float32),
        ],
    )(query, positive_key)

    out = pl.pallas_call(
        _loss_block_kernel,
        grid=(N // BLOCK,),
        in_specs=[
            pl.BlockSpec((BLOCK, D), lambda i: (i, 0)),
            pl.BlockSpec((N, D), lambda i: (0, 0)),
            pl.BlockSpec((BLOCK, D), lambda i: (i, 0)),
        ],
        out_specs=pl.BlockSpec((1, 1, 1), lambda i: (i, 0, 0)),
        out_shape=jax.ShapeDtypeStruct((N // BLOCK, 1, 1), jnp.float32),
        compiler_params=pltpu.CompilerParams(
            dimension_semantics=("parallel",),
        ),
    )(qn, pn, pn)
    return jnp.sum(out) / N


# speedup vs baseline: 1.0761x; 1.0055x over previous
import functools

import jax
import jax.numpy as jnp
from jax.experimental import pallas as pl
from jax.experimental.pallas import tpu as pltpu

N = 4096
D = 128
TEMP = 0.1
K = max(1, int(0.5 * (N - 1)))
BLOCK = 1024
N_ITERS = 13
LOG2E = 1.4426950408889634
LN2 = 0.6931471805599453
LO0 = -1.00005 * LOG2E / TEMP
HI0 = 1.00005 * LOG2E / TEMP


def _normalize_kernel(q_ref, p_ref, qn_ref, pn_ref):
    q = q_ref[...]
    p = p_ref[...]
    qs = (LOG2E / TEMP) / jnp.maximum(
        jnp.sqrt(jnp.sum(q * q, axis=1, keepdims=True)), 1e-12)
    ps = 1.0 / jnp.maximum(
        jnp.sqrt(jnp.sum(p * p, axis=1, keepdims=True)), 1e-12)
    qn_ref[...] = q * qs
    pn_ref[...] = p * ps


def _loss_block_kernel(q_ref, p_ref, pblk_ref, out_ref):
    qb = q_ref[...]
    pf = p_ref[...]

    logits = jax.lax.dot_general(
        qb, pf, dimension_numbers=(((1,), (1,)), ((), ())),
        preferred_element_type=jnp.float32,
    )

    pos = jnp.sum(qb * pblk_ref[...], axis=1, keepdims=True)

    kf = jnp.float32(K)

    def bs_body(_, carry):
        lo, hi = carry
        mid = 0.5 * (lo + hi)
        cnt = jnp.sum((logits < mid).astype(jnp.float32), axis=1, keepdims=True)
        cnt = cnt - (pos < mid).astype(jnp.float32)
        ge = cnt >= kf
        return jnp.where(ge, lo, mid), jnp.where(ge, mid, hi)

    lo = jnp.full((BLOCK, 1), LO0, jnp.float32)
    hi = jnp.full((BLOCK, 1), HI0, jnp.float32)
    lo, hi = jax.lax.fori_loop(0, N_ITERS - 1, bs_body, (lo, hi))

    mid = 0.5 * (lo + hi)
    below = logits < mid
    e = jnp.where(below, jnp.exp2(logits), 0.0)
    es = jnp.sum(e, axis=1, keepdims=True)
    c = jnp.sum(below.astype(jnp.float32), axis=1, keepdims=True)
    c = c - (pos < mid).astype(jnp.float32)
    that = mid + jnp.where(c >= kf, -0.125, 0.125) * (hi - lo)
    s = (es - (pos < mid).astype(jnp.float32) * jnp.exp2(pos)
         + (kf - c) * jnp.exp2(that) + jnp.exp2(pos))
    losses = -pos * LN2 + jnp.log(s)

    out_ref[...] = jnp.sum(losses).reshape(1, 1, 1)


@jax.jit
def kernel(query, positive_key):
    qn, pn = pl.pallas_call(
        _normalize_kernel,
        in_specs=[
            pl.BlockSpec((N, D), lambda: (0, 0)),
            pl.BlockSpec((N, D), lambda: (0, 0)),
        ],
        out_specs=[
            pl.BlockSpec((N, D), lambda: (0, 0)),
            pl.BlockSpec((N, D), lambda: (0, 0)),
        ],
        out_shape=[
            jax.ShapeDtypeStruct((N, D), jnp.float32),
            jax.ShapeDtypeStruct((N, D), jnp.float32),
        ],
    )(query, positive_key)

    out = pl.pallas_call(
        _loss_block_kernel,
        grid=(N // BLOCK,),
        in_specs=[
            pl.BlockSpec((BLOCK, D), lambda i: (i, 0)),
            pl.BlockSpec((N, D), lambda i: (0, 0)),
            pl.BlockSpec((BLOCK, D), lambda i: (i, 0)),
        ],
        out_specs=pl.BlockSpec((1, 1, 1), lambda i: (i, 0, 0)),
        out_shape=jax.ShapeDtypeStruct((N // BLOCK, 1, 1), jnp.float32),
        compiler_params=pltpu.CompilerParams(
            dimension_semantics=("parallel",),
        ),
    )(qn, pn, pn)
    return jnp.sum(out) / N


# speedup vs baseline: 1.2871x; 1.1961x over previous
import functools

import jax
import jax.numpy as jnp
from jax.experimental import pallas as pl
from jax.experimental.pallas import tpu as pltpu

N = 4096
D = 128
TEMP = 0.1
K = max(1, int(0.5 * (N - 1)))
BLOCK = 1024
N_ITERS = 10
LOG2E = 1.4426950408889634
LN2 = 0.6931471805599453
LO0 = -1.00005 * LOG2E / TEMP
HI0 = 1.00005 * LOG2E / TEMP


def _normalize_kernel(q_ref, p_ref, qn_ref, pn_ref, psum_ref, gram_ref):
    q = q_ref[...]
    p = p_ref[...]
    qs = (LOG2E / TEMP) / jnp.maximum(
        jnp.sqrt(jnp.sum(q * q, axis=1, keepdims=True)), 1e-12)
    ps = 1.0 / jnp.maximum(
        jnp.sqrt(jnp.sum(p * p, axis=1, keepdims=True)), 1e-12)
    pn = p * ps
    qn_ref[...] = q * qs
    pn_ref[...] = pn
    psum_ref[...] = jnp.sum(pn, axis=0, keepdims=True)
    gram_ref[...] = jax.lax.dot_general(
        pn, pn, dimension_numbers=(((0,), (0,)), ((), ())),
        preferred_element_type=jnp.float32,
    )


def _loss_block_kernel(q_ref, p_ref, pblk_ref, psum_ref, gram_ref, out_ref):
    qb = q_ref[...]
    pf = p_ref[...]

    logits = jax.lax.dot_general(
        qb, pf, dimension_numbers=(((1,), (1,)), ((), ())),
        preferred_element_type=jnp.float32,
    )

    pos = jnp.sum(qb * pblk_ref[...], axis=1, keepdims=True)

    kf = jnp.float32(K)

    inv_n = 1.0 / N
    mu = jnp.sum(qb * psum_ref[...], axis=1, keepdims=True) * inv_n
    qg = jax.lax.dot_general(
        qb, gram_ref[...], dimension_numbers=(((1,), (0,)), ((), ())),
        preferred_element_type=jnp.float32,
    )
    ex2 = jnp.sum(qg * qb, axis=1, keepdims=True) * inv_n
    sig = jnp.sqrt(jnp.maximum(ex2 - mu * mu, 0.0))

    def bs_body(_, carry):
        lo, hi = carry
        mid = 0.5 * (lo + hi)
        cnt = jnp.sum((logits < mid).astype(jnp.float32), axis=1, keepdims=True)
        cnt = cnt - (pos < mid).astype(jnp.float32)
        ge = cnt >= kf
        return jnp.where(ge, lo, mid), jnp.where(ge, mid, hi)

    lo = jnp.maximum(mu - 1.35 * sig - 1e-3, LO0)
    hi = jnp.minimum(mu + 1.35 * sig + 1e-3, HI0)
    lo, hi = jax.lax.fori_loop(0, N_ITERS - 1, bs_body, (lo, hi))

    mid = 0.5 * (lo + hi)
    below = logits < mid
    e = jnp.where(below, jnp.exp2(logits), 0.0)
    es = jnp.sum(e, axis=1, keepdims=True)
    c = jnp.sum(below.astype(jnp.float32), axis=1, keepdims=True)
    c = c - (pos < mid).astype(jnp.float32)
    that = mid + jnp.where(c >= kf, -0.125, 0.125) * (hi - lo)
    s = (es - (pos < mid).astype(jnp.float32) * jnp.exp2(pos)
         + (kf - c) * jnp.exp2(that) + jnp.exp2(pos))
    losses = -pos * LN2 + jnp.log(s)

    out_ref[...] = jnp.sum(losses).reshape(1, 1, 1)


@jax.jit
def kernel(query, positive_key):
    qn, pn, psum, gram = pl.pallas_call(
        _normalize_kernel,
        in_specs=[
            pl.BlockSpec((N, D), lambda: (0, 0)),
            pl.BlockSpec((N, D), lambda: (0, 0)),
        ],
        out_specs=[
            pl.BlockSpec((N, D), lambda: (0, 0)),
            pl.BlockSpec((N, D), lambda: (0, 0)),
            pl.BlockSpec((1, D), lambda: (0, 0)),
            pl.BlockSpec((D, D), lambda: (0, 0)),
        ],
        out_shape=[
            jax.ShapeDtypeStruct((N, D), jnp.float32),
            jax.ShapeDtypeStruct((N, D), jnp.float32),
            jax.ShapeDtypeStruct((1, D), jnp.float32),
            jax.ShapeDtypeStruct((D, D), jnp.float32),
        ],
    )(query, positive_key)

    out = pl.pallas_call(
        _loss_block_kernel,
        grid=(N // BLOCK,),
        in_specs=[
            pl.BlockSpec((BLOCK, D), lambda i: (i, 0)),
            pl.BlockSpec((N, D), lambda i: (0, 0)),
            pl.BlockSpec((BLOCK, D), lambda i: (i, 0)),
            pl.BlockSpec((1, D), lambda i: (0, 0)),
            pl.BlockSpec((D, D), lambda i: (0, 0)),
        ],
        out_specs=pl.BlockSpec((1, 1, 1), lambda i: (i, 0, 0)),
        out_shape=jax.ShapeDtypeStruct((N // BLOCK, 1, 1), jnp.float32),
        compiler_params=pltpu.CompilerParams(
            dimension_semantics=("parallel",),
        ),
    )(qn, pn, pn, psum, gram)
    return jnp.sum(out) / N
